# Initial kernel scaffold; baseline (speedup 1.0000x reference)
import jax
import jax.numpy as jnp
from jax.experimental import pallas as pl


def kernel(item_ids, title_tokens, table_id, table_text):
    raise NotImplementedError("write your pallas kernel here")


# speedup vs baseline: 15.7212x; 15.7212x over previous
import functools

import jax
import jax.numpy as jnp
from jax import lax
from jax.experimental import pallas as pl
from jax.experimental.pallas import tpu as pltpu
from jax.experimental.pallas import tpu_sc as plsc

_B = 16384
_L = 20
_EMB = 32
_NC = 2
_NS = 16
_NW = _NC * _NS
_BPW = _B // _NW
_CH = 64
_NCH = _BPW // _CH
_PAIRS = _NCH // 2


@functools.partial(
    pl.kernel,
    out_type=jax.ShapeDtypeStruct((_B, 2 * _EMB), jnp.float32),
    scratch_types=[
        pltpu.VMEM((4, 128), jnp.int32),
        pltpu.VMEM((_NCH, _L, _CH), jnp.int32),
        pltpu.VMEM((_BPW, _EMB), jnp.float32),
        pltpu.VMEM((_L * _CH, _EMB), jnp.float32),
        pltpu.VMEM((_L * _CH, _EMB), jnp.float32),
        pltpu.VMEM((_CH, 2 * _EMB), jnp.float32),
        pltpu.VMEM((1, _EMB), jnp.float32),
        pltpu.SemaphoreType.DMA,
        pltpu.SemaphoreType.DMA,
        pltpu.SemaphoreType.DMA,
    ],
    mesh=plsc.VectorSubcoreMesh(core_axis_name="c", subcore_axis_name="s"),
    compiler_params=pltpu.CompilerParams(use_tc_tiling_on_sc=False),
)
def _sc_item_model(ids_hbm, tok_hbm, tid_hbm, ttx_hbm, out_hbm,
                   ids_v, tok_v, idrows_v, g0_v, g1_v, outc_v, row0_v,
                   sem_id, sem_a, sem_b):
    c = lax.axis_index("c")
    s = lax.axis_index("s")
    w = s * _NC + c
    base = w * _BPW

    pltpu.sync_copy(ids_hbm.at[w], ids_v)
    pltpu.sync_copy(tok_hbm.at[w], tok_v)
    pltpu.sync_copy(ttx_hbm.at[pl.ds(0, 1), :], row0_v)

    for q in range(4):
        pltpu.async_copy(tid_hbm.at[ids_v.at[q]],
                         idrows_v.at[pl.ds(q * 128, 128), :], sem_id)

    def fire(cc, gbuf, sem):
        for l in range(_L):
            pltpu.async_copy(ttx_hbm.at[tok_v.at[cc, l]],
                             gbuf.at[pl.ds(l * _CH, _CH), :], sem)

    def drain(gbuf, sem):
        pltpu.make_async_copy(ttx_hbm.at[pl.ds(0, _L * _CH), :], gbuf, sem).wait()

    fire(0, g0_v, sem_a)
    pltpu.make_async_copy(tid_hbm.at[pl.ds(0, _BPW), :], idrows_v, sem_id).wait()

    row0_lo = row0_v[0, pl.ds(0, 16)]
    row0_hi = row0_v[0, pl.ds(16, 16)]

    def compute(cc, gbuf):
        def group_body(g, carry):
            cnt = jnp.zeros((16,), jnp.int32)
            for l in range(_L):
                t = tok_v[cc, l, pl.ds(g * 16, 16)]
                cnt = cnt + jnp.minimum(t, 1)
            cntf = cnt.astype(jnp.float32)
            rec_vec = 1.0 / jnp.maximum(cntf, 1.0)
            pad_vec = jnp.float32(_L) - cntf

            for j in range(16):
                b = g * 16 + j
                r = cc * _CH + b
                rec = jnp.full((16,), rec_vec[j], jnp.float32)
                pad = jnp.full((16,), pad_vec[j], jnp.float32)
                outc_v[b, pl.ds(0, 16)] = idrows_v[r, pl.ds(0, 16)]
                outc_v[b, pl.ds(16, 16)] = idrows_v[r, pl.ds(16, 16)]
                tlo = gbuf[b, pl.ds(0, 16)]
                thi = gbuf[b, pl.ds(16, 16)]
                for l in range(1, _L):
                    tlo = tlo + gbuf[l * _CH + b, pl.ds(0, 16)]
                    thi = thi + gbuf[l * _CH + b, pl.ds(16, 16)]
                outc_v[b, pl.ds(32, 16)] = (tlo - pad * row0_lo) * rec
                outc_v[b, pl.ds(48, 16)] = (thi - pad * row0_hi) * rec
            return carry

        lax.fori_loop(0, _CH // 16, group_body, 0)
        pltpu.sync_copy(outc_v, out_hbm.at[pl.ds(base + cc * _CH, _CH), :])

    def pair_body(i, carry):
        cc0 = 2 * i
        fire(cc0 + 1, g1_v, sem_b)
        drain(g0_v, sem_a)
        compute(cc0, g0_v)

        @pl.when(i < _PAIRS - 1)
        def _():
            fire(cc0 + 2, g0_v, sem_a)

        drain(g1_v, sem_b)
        compute(cc0 + 1, g1_v)
        return carry

    lax.fori_loop(0, _PAIRS, pair_body, 0)


@jax.jit
def kernel(item_ids, title_tokens, table_id, table_text):
    ids = item_ids.astype(jnp.int32).reshape(_NW, 4, 128)
    tok = (title_tokens.astype(jnp.int32)
           .reshape(_NW, _NCH, _CH, _L)
           .transpose(0, 1, 3, 2))
    return _sc_item_model(ids, tok, table_id, table_text)
